# baseline (speedup 1.0000x reference)
import functools

import jax
import jax.numpy as jnp
from jax import lax
from jax.experimental import pallas as pl
from jax.experimental.pallas import tpu as pltpu
from jax.experimental.pallas import tpu_sc as plsc

ROWS = 32768
COLS = 64
L = 16

RG = 8
NGRP = COLS // RG
KCH = 4
C_CHUNK = ROWS // KCH
NVREG = C_CHUNK // L

_mesh = plsc.VectorSubcoreMesh(core_axis_name="c", subcore_axis_name="s",
                               num_cores=2, num_subcores=16)
_params = pltpu.CompilerParams(use_tc_tiling_on_sc=True,
                               needs_layout_passes=False)

_GDN = lax.GatherDimensionNumbers(
    offset_dims=(), collapsed_slice_dims=(0,), start_index_map=(0,))


def _bcast_last(y):
    idx = jnp.full((L, 1), L - 1, jnp.int32)
    return lax.gather(y, idx, _GDN, (1,),
                      mode=lax.GatherScatterMode.PROMISE_IN_BOUNDS)


NPAN = 8
C_PAN = C_CHUNK // NPAN
NV_PAN = C_PAN // L


def _sc_body(x_hbm, out_hbm, buf, tv, at_buf, shared, in_sems, out_sems):
    c = lax.axis_index("c")
    s = lax.axis_index("s")
    g_local = s // KCH
    k = s % KCH
    r0 = (c * (NGRP // 2) + g_local) * RG
    c0 = k * C_CHUNK

    in_copies = []
    for p in range(NPAN):
        cp = pltpu.make_async_copy(
            x_hbm.at[pl.ds(r0, RG), pl.ds(c0 + p * C_PAN, C_PAN)],
            buf.at[:, pl.ds(p * C_PAN, C_PAN)],
            in_sems.at[p])
        cp.start()
        in_copies.append(cp)

    zero = jnp.zeros((L,), jnp.float32)
    tot = (zero,) * RG
    for p in range(NPAN):
        in_copies[p].wait()

        def body_a(j, accs):
            for u in range(2):
                accs = tuple(
                    accs[r] + buf[r, pl.ds(p * C_PAN + (j * 2 + u) * L, L)]
                    for r in range(RG))
            return accs

        tot = lax.fori_loop(0, NV_PAN // 2, body_a, tot)

    for r in range(RG):
        tv[r, pl.ds(0, L)] = jnp.full((L,), jnp.sum(tot[r]), jnp.float32)

    pltpu.sync_copy(tv, shared.at[s])
    plsc.subcore_barrier()
    pltpu.sync_copy(shared.at[pl.ds(g_local * KCH, KCH - 1)], at_buf)

    offs = [jnp.full((L,), -1.0, jnp.float32) for _ in range(RG)]
    for kp in range(KCH - 1):
        m = (kp < k).astype(jnp.float32)
        for r in range(RG):
            offs[r] = offs[r] + at_buf[kp, r, pl.ds(0, L)] * m

    carrys = tuple(offs)
    out_copies = []
    for p in range(NPAN):

        def body_b(j, cs):
            for u in range(2):
                new = []
                for r in range(RG):
                    x = buf[r, pl.ds(p * C_PAN + (j * 2 + u) * L, L)]
                    y = plsc.cumsum(x) + cs[r]
                    buf[r, pl.ds(p * C_PAN + (j * 2 + u) * L, L)] = y
                    new.append(_bcast_last(y))
                cs = tuple(new)
            return cs

        carrys = lax.fori_loop(0, NV_PAN // 2, body_b, carrys)
        cp = pltpu.make_async_copy(
            buf.at[:, pl.ds(p * C_PAN, C_PAN)],
            out_hbm.at[pl.ds(r0, RG), pl.ds(c0 + p * C_PAN, C_PAN)],
            out_sems.at[p])
        cp.start()
        out_copies.append(cp)

    for cp in out_copies:
        cp.wait()


def _build(interpret=False):
    return pl.kernel(
        _sc_body,
        out_type=jax.ShapeDtypeStruct((COLS, ROWS), jnp.float32),
        mesh=_mesh,
        scratch_types=[
            pltpu.VMEM((RG, C_CHUNK), jnp.float32),
            pltpu.VMEM((RG, 128), jnp.float32),
            pltpu.VMEM((KCH - 1, RG, 128), jnp.float32),
            pltpu.MemorySpace.VMEM_SHARED((16, RG, 128), jnp.float32),
            pltpu.SemaphoreType.DMA((NPAN,)),
            pltpu.SemaphoreType.DMA((NPAN,)),
        ],
        compiler_params=_params,
        interpret=interpret,
    )


_sc_cumsum_t = _build()


def kernel(mask_i):
    return _sc_cumsum_t(mask_i.T).T


# speedup vs baseline: 1.1114x; 1.1114x over previous
import functools

import jax
import jax.numpy as jnp
from jax import lax
from jax.experimental import pallas as pl
from jax.experimental.pallas import tpu as pltpu
from jax.experimental.pallas import tpu_sc as plsc

ROWS = 32768
COLS = 64
L = 16

RG = 8
NGRP = COLS // RG
KCH = 4
C_CHUNK = ROWS // KCH
NVREG = C_CHUNK // L

_mesh = plsc.VectorSubcoreMesh(core_axis_name="c", subcore_axis_name="s",
                               num_cores=2, num_subcores=16)
_params = pltpu.CompilerParams(use_tc_tiling_on_sc=True,
                               needs_layout_passes=False)

_GDN = lax.GatherDimensionNumbers(
    offset_dims=(), collapsed_slice_dims=(0,), start_index_map=(0,))


def _bcast_last(y):
    idx = jnp.full((L, 1), L - 1, jnp.int32)
    return lax.gather(y, idx, _GDN, (1,),
                      mode=lax.GatherScatterMode.PROMISE_IN_BOUNDS)


NPAN = 8
C_PAN = C_CHUNK // NPAN
NV_PAN = C_PAN // L


def _sc_body(x_hbm, out_hbm, buf, tv, at_buf, shared, in_sems, out_sems):
    c = lax.axis_index("c")
    s = lax.axis_index("s")
    g_local = s // KCH
    k = s % KCH
    r0 = (c * (NGRP // 2) + g_local) * RG
    c0 = k * C_CHUNK

    in_copies = []
    for p in range(NPAN):
        cp = pltpu.make_async_copy(
            x_hbm.at[pl.ds(r0, RG), pl.ds(c0 + p * C_PAN, C_PAN)],
            buf.at[:, pl.ds(p * C_PAN, C_PAN)],
            in_sems.at[p])
        cp.start()
        in_copies.append(cp)

    zero = jnp.zeros((L,), jnp.float32)
    tot = (zero,) * RG
    for p in range(NPAN):
        in_copies[p].wait()

        def body_a(j, accs):
            return tuple(accs[r] + buf[r, pl.ds(p * C_PAN + j * L, L)]
                         for r in range(RG))

        tot = lax.fori_loop(0, NV_PAN, body_a, tot)

    for r in range(RG):
        tv[r, pl.ds(0, L)] = jnp.full((L,), jnp.sum(tot[r]), jnp.float32)

    pltpu.sync_copy(tv, shared.at[s])
    plsc.subcore_barrier()
    pltpu.sync_copy(shared.at[pl.ds(g_local * KCH, KCH - 1)], at_buf)

    offs = [jnp.full((L,), -1.0, jnp.float32) for _ in range(RG)]
    for kp in range(KCH - 1):
        m = (kp < k).astype(jnp.float32)
        for r in range(RG):
            offs[r] = offs[r] + at_buf[kp, r, pl.ds(0, L)] * m

    carrys = tuple(offs)
    out_copies = []
    for p in range(NPAN):

        def body_b(j, cs):
            new = []
            for r in range(RG):
                x = buf[r, pl.ds(p * C_PAN + j * L, L)]
                y = plsc.cumsum(x) + cs[r]
                buf[r, pl.ds(p * C_PAN + j * L, L)] = y
                new.append(_bcast_last(y))
            return tuple(new)

        carrys = lax.fori_loop(0, NV_PAN, body_b, carrys)
        cp = pltpu.make_async_copy(
            buf.at[:, pl.ds(p * C_PAN, C_PAN)],
            out_hbm.at[pl.ds(r0, RG), pl.ds(c0 + p * C_PAN, C_PAN)],
            out_sems.at[p])
        cp.start()
        out_copies.append(cp)

    for cp in out_copies:
        cp.wait()


def _build(interpret=False):
    return pl.kernel(
        _sc_body,
        out_type=jax.ShapeDtypeStruct((COLS, ROWS), jnp.float32),
        mesh=_mesh,
        scratch_types=[
            pltpu.VMEM((RG, C_CHUNK), jnp.float32),
            pltpu.VMEM((RG, 128), jnp.float32),
            pltpu.VMEM((KCH - 1, RG, 128), jnp.float32),
            pltpu.MemorySpace.VMEM_SHARED((16, RG, 128), jnp.float32),
            pltpu.SemaphoreType.DMA((NPAN,)),
            pltpu.SemaphoreType.DMA((NPAN,)),
        ],
        compiler_params=_params,
        interpret=interpret,
    )


_sc_cumsum_t = _build()


def kernel(mask_i):
    return _sc_cumsum_t(mask_i.T).T
